# Initial kernel scaffold; baseline (speedup 1.0000x reference)
import functools

import jax
import jax.numpy as jnp
from jax import lax
from jax.experimental import pallas as pl
from jax.experimental.pallas import tpu as pltpu
from jax.experimental.pallas import tpu_sc as plsc

_D = 32
_G = 128
_NG = 8
_C = _G * _NG


@functools.cache
def _make_lookup(B: int, V: int):
    info = plsc.get_sparse_core_info()
    nc, ns = info.num_cores, info.num_subcores
    nw = nc * ns
    b_per_w = B // nw
    n_chunks = b_per_w // _C
    assert b_per_w % _C == 0

    mesh = plsc.VectorSubcoreMesh(core_axis_name="c", subcore_axis_name="s")

    @functools.partial(
        pl.kernel,
        mesh=mesh,
        out_type=jax.ShapeDtypeStruct((B, _D), jnp.float32),
        scratch_types=[
            pltpu.VMEM((_NG, _G), jnp.int32),
            pltpu.VMEM((_C, _D), jnp.float32),
            pltpu.SemaphoreType.DMA,
        ],
    )
    def lookup(table_hbm, idx_hbm, out_hbm, idx_v, rows_v, sem):
        wid = lax.axis_index("s") * nc + lax.axis_index("c")
        base = wid * b_per_w

        @pl.loop(0, n_chunks)
        def _chunk(ci):
            row0 = base + ci * _C
            pltpu.sync_copy(idx_hbm.at[pl.ds(row0 // _G, _NG)], idx_v)
            copies = [
                pltpu.async_copy(
                    table_hbm.at[idx_v.at[j]],
                    rows_v.at[pl.ds(j * _G, _G)],
                    sem,
                )
                for j in range(_NG)
            ]
            for cp in copies:
                cp.wait()
            pltpu.sync_copy(rows_v, out_hbm.at[pl.ds(row0, _C)])

    return lookup


def kernel(indices, weight):
    B = indices.size
    idx2d = indices.reshape(B // _G, _G).astype(jnp.int32)
    out = _make_lookup(B, weight.shape[0])(weight, idx2d)
    return out.reshape(indices.shape + (weight.shape[1],))


# speedup vs baseline: 1.0946x; 1.0946x over previous
import functools

import jax
import jax.numpy as jnp
from jax import lax
from jax.experimental import pallas as pl
from jax.experimental.pallas import tpu as pltpu
from jax.experimental.pallas import tpu_sc as plsc

_D = 32
_G = 128
_NG = 8
_C = _G * _NG


@functools.cache
def _make_lookup(B: int, V: int):
    info = plsc.get_sparse_core_info()
    nc, ns = info.num_cores, info.num_subcores
    nw = nc * ns
    b_per_w = B // nw
    n_chunks = b_per_w // _C
    assert b_per_w % _C == 0

    mesh = plsc.VectorSubcoreMesh(core_axis_name="c", subcore_axis_name="s")

    @functools.partial(
        pl.kernel,
        mesh=mesh,
        compiler_params=pltpu.CompilerParams(use_tc_tiling_on_sc=False),
        out_type=jax.ShapeDtypeStruct((B, _D), jnp.float32),
        scratch_types=[
            pltpu.VMEM((_NG, _G), jnp.int32),
            pltpu.VMEM((_C, _D), jnp.float32),
            pltpu.SemaphoreType.DMA,
        ],
    )
    def lookup(table_hbm, idx_hbm, out_hbm, idx_v, rows_v, sem):
        wid = lax.axis_index("s") * nc + lax.axis_index("c")
        base = wid * b_per_w

        @pl.loop(0, n_chunks)
        def _chunk(ci):
            row0 = base + ci * _C
            irow0 = pl.multiple_of(row0 // _G, 8)
            pltpu.sync_copy(idx_hbm.at[pl.ds(irow0, _NG)], idx_v)
            copies = [
                pltpu.async_copy(
                    table_hbm.at[idx_v.at[j]],
                    rows_v.at[pl.ds(j * _G, _G)],
                    sem,
                )
                for j in range(_NG)
            ]
            for cp in copies:
                cp.wait()
            pltpu.sync_copy(rows_v, out_hbm.at[pl.ds(row0, _C)])

    return lookup


def kernel(indices, weight):
    B = indices.size
    idx2d = indices.reshape(B // _G, _G).astype(jnp.int32)
    out = _make_lookup(B, weight.shape[0])(weight, idx2d)
    return out.reshape(indices.shape + (weight.shape[1],))


# speedup vs baseline: 1.1132x; 1.0170x over previous
import functools

import jax
import jax.numpy as jnp
from jax import lax
from jax.experimental import pallas as pl
from jax.experimental.pallas import tpu as pltpu
from jax.experimental.pallas import tpu_sc as plsc

_D = 32
_C = 1280


@functools.cache
def _make_lookup(B: int, V: int):
    info = plsc.get_sparse_core_info()
    nc, ns = info.num_cores, info.num_subcores
    nw = nc * ns
    b_per_w = B // nw
    n_chunks = b_per_w // _C
    assert b_per_w % _C == 0 and n_chunks % 2 == 0

    mesh = plsc.VectorSubcoreMesh(core_axis_name="c", subcore_axis_name="s")

    @functools.partial(
        pl.kernel,
        mesh=mesh,
        compiler_params=pltpu.CompilerParams(use_tc_tiling_on_sc=False),
        out_type=jax.ShapeDtypeStruct((B, _D), jnp.float32),
        scratch_types=[
            pltpu.VMEM((_C,), jnp.int32),
            pltpu.VMEM((_C,), jnp.int32),
            pltpu.VMEM((_C, _D), jnp.float32),
            pltpu.VMEM((_C, _D), jnp.float32),
            pltpu.SemaphoreType.DMA,
            pltpu.SemaphoreType.DMA,
            pltpu.SemaphoreType.DMA,
            pltpu.SemaphoreType.DMA,
            pltpu.SemaphoreType.DMA,
            pltpu.SemaphoreType.DMA,
        ],
    )
    def lookup(table_hbm, idx_hbm, out_hbm,
               idx0, idx1, rows0, rows1,
               si0, si1, sg0, sg1, so0, so1):
        wid = lax.axis_index("s") * nc + lax.axis_index("c")
        base = wid * b_per_w
        idx_b = (idx0, idx1)
        rows_b = (rows0, rows1)
        si = (si0, si1)
        sg = (sg0, sg1)
        so = (so0, so1)

        def row0_of(ci):
            return pl.multiple_of(base + ci * _C, 8)

        pltpu.async_copy(idx_hbm.at[pl.ds(row0_of(0), _C)], idx0, si0)

        def stage(ci, p):
            q = 1 - p

            @pl.when(ci >= 2)
            def _():
                pltpu.make_async_copy(
                    rows_b[p], out_hbm.at[pl.ds(row0_of(ci - 2), _C)], so[p]
                ).wait()

            pltpu.make_async_copy(
                idx_hbm.at[pl.ds(row0_of(ci), _C)], idx_b[p], si[p]
            ).wait()
            pltpu.async_copy(table_hbm.at[idx_b[p]], rows_b[p], sg[p])

            @pl.when(ci >= 1)
            def _():
                pltpu.make_async_copy(
                    table_hbm.at[idx_b[q]], rows_b[q], sg[q]
                ).wait()
                pltpu.async_copy(
                    rows_b[q], out_hbm.at[pl.ds(row0_of(ci - 1), _C)], so[q]
                )

            @pl.when(ci + 1 < n_chunks)
            def _():
                pltpu.async_copy(
                    idx_hbm.at[pl.ds(row0_of(ci + 1), _C)], idx_b[q], si[q]
                )

        @pl.loop(0, n_chunks, step=2)
        def _pair(ci0):
            stage(ci0, 0)
            stage(ci0 + 1, 1)

        last = n_chunks - 1
        pltpu.make_async_copy(table_hbm.at[idx1], rows1, sg1).wait()
        pltpu.async_copy(rows1, out_hbm.at[pl.ds(row0_of(last), _C)], so1)
        pltpu.make_async_copy(
            rows0, out_hbm.at[pl.ds(row0_of(last - 1), _C)], so0
        ).wait()
        pltpu.make_async_copy(
            rows1, out_hbm.at[pl.ds(row0_of(last), _C)], so1
        ).wait()

    return lookup


def kernel(indices, weight):
    B = indices.size
    idx_flat = indices.reshape(B).astype(jnp.int32)
    out = _make_lookup(B, weight.shape[0])(weight, idx_flat)
    return out.reshape(indices.shape + (weight.shape[1],))
